# baseline (speedup 1.0000x reference)
import functools

import jax
import jax.numpy as jnp
from jax import lax
from jax.experimental import pallas as pl
from jax.experimental.pallas import tpu as pltpu
from jax.experimental.pallas import tpu_sc as plsc

N = 10000
E = 320000
D = 128
H = 16
C = 64

NUM_CORES = 2
NUM_TILES = 16
NPAD = 10240
HALF = NPAD // NUM_CORES
ROWS_PER_TILE = HALF // NUM_TILES
ZERO_ROW = N
ACC_ROWS = HALF + 16
DUMMY_DST = HALF

EDGES_PER_TILE = E // NUM_TILES
CHUNK = 128
K_CHUNKS = 8
GROUP_E = K_CHUNKS * CHUNK
GROUPS = 20
NCHUNK = GROUPS * K_CHUNKS
EPT_PAD = NCHUNK * CHUNK


def _prep_body(x_ref, w1_ref, src_ref, dst_ref, y_ref, srcg_ref, d0_ref, d1_ref):
    x = x_ref[...]
    w1 = w1_ref[...]
    y_ref[...] = lax.dot_general(x, w1, (((1,), (1,)), ((), ())),
                                 preferred_element_type=jnp.float32,
                                 precision=lax.Precision.HIGHEST)
    src = src_ref[...]
    dst = dst_ref[...]
    srcg_ref[...] = jnp.where(src == dst, ZERO_ROW, src)
    d0_ref[...] = jnp.where(dst < HALF, dst, DUMMY_DST)
    d1_ref[...] = jnp.where(dst >= HALF, dst - HALF, DUMMY_DST)


_prep_call = pl.pallas_call(
    _prep_body,
    out_shape=(
        jax.ShapeDtypeStruct((NPAD, H), jnp.float32),
        jax.ShapeDtypeStruct((E // 128, 128), jnp.int32),
        jax.ShapeDtypeStruct((E // 128, 128), jnp.int32),
        jax.ShapeDtypeStruct((E // 128, 128), jnp.int32),
    ),
)


def _sc_step_body(h_hbm, srcg_hbm, dstl_hbm, out_hbm, acc, sidx, didx,
                  rows, sem):
    c = lax.axis_index("c")
    s = lax.axis_index("s")
    base = c * HALF + s * ROWS_PER_TILE
    pltpu.sync_copy(h_hbm.at[pl.ds(base, ROWS_PER_TILE)],
                    acc.at[pl.ds(s * ROWS_PER_TILE, ROWS_PER_TILE)])
    pltpu.sync_copy(srcg_hbm.at[s], sidx)
    pltpu.sync_copy(dstl_hbm.at[c, s], didx)
    plsc.subcore_barrier()

    def body(g, carry):
        sl = pl.ds(g * GROUP_E, GROUP_E)
        pltpu.async_copy(h_hbm.at[sidx.at[sl]], rows, sem).wait()
        pltpu.sync_copy(rows, acc.at[didx.at[sl]], add=True)
        return carry

    lax.fori_loop(0, GROUPS, body, 0)
    plsc.subcore_barrier()
    pltpu.sync_copy(acc.at[pl.ds(s * ROWS_PER_TILE, ROWS_PER_TILE)],
                    out_hbm.at[pl.ds(base, ROWS_PER_TILE)])


@functools.cache
def _get_sc_step():
    return pl.kernel(
        _sc_step_body,
        out_type=jax.ShapeDtypeStruct((NPAD, H), jnp.float32),
        mesh=plsc.VectorSubcoreMesh(core_axis_name="c", subcore_axis_name="s",
                                    num_cores=NUM_CORES, num_subcores=NUM_TILES),
        scratch_types=[
            pltpu.VMEM_SHARED((ACC_ROWS, H), jnp.float32),
            pltpu.VMEM((EPT_PAD,), jnp.int32),
            pltpu.VMEM((EPT_PAD,), jnp.int32),
            pltpu.VMEM((GROUP_E, H), jnp.float32),
            pltpu.SemaphoreType.DMA,
        ],
        compiler_params=pltpu.CompilerParams(use_tc_tiling_on_sc=False),
    )


_SELU_SCALE = 1.0507009873554805
_SELU_ALPHA = 1.6732632423543772


def _bn_body(h_ref, b1_ref, g_ref, bt_ref, o_ref):
    h = h_ref[...]
    mask = (lax.broadcasted_iota(jnp.int32, (NPAD, 1), 0) < N).astype(jnp.float32)
    hb = (h + b1_ref[...]) * mask
    mean = jnp.sum(hb, axis=0, keepdims=True) / N
    ctr = (hb - mean) * mask
    var = jnp.sum(ctr * ctr, axis=0, keepdims=True) / N
    z = (hb - mean) * lax.rsqrt(var + 1e-5) * g_ref[...] + bt_ref[...]
    act = _SELU_SCALE * jnp.where(z > 0, z, _SELU_ALPHA * (jnp.exp(z) - 1.0))
    o_ref[...] = act * mask


_bn_call = pl.pallas_call(
    _bn_body,
    out_shape=jax.ShapeDtypeStruct((NPAD, H), jnp.float32),
)


def _out_body(h_ref, w2_ref, b2_ref, o_ref):
    h = h_ref[...]
    logits = lax.dot_general(h, w2_ref[...], (((1,), (1,)), ((), ())),
                             preferred_element_type=jnp.float32) + b2_ref[...]
    m = jnp.max(logits, axis=1, keepdims=True)
    e = jnp.exp(logits - m)
    p = e / jnp.sum(e, axis=1, keepdims=True)
    o_ref[...] = p[:N, :]


_out_call = pl.pallas_call(
    _out_body,
    out_shape=jax.ShapeDtypeStruct((N, C), jnp.float32),
)


def _to_slabs(a, fill):
    a = a.reshape(NUM_TILES, EDGES_PER_TILE)
    a = jnp.pad(a, ((0, 0), (0, EPT_PAD - EDGES_PER_TILE)), constant_values=fill)
    return a


def kernel(x, edge_index, W1, b1, gamma, beta, W2, b2):
    x_pad = jnp.pad(x, ((0, NPAD - N), (0, 0)))
    src2d = edge_index[0].reshape(E // 128, 128)
    dst2d = edge_index[1].reshape(E // 128, 128)
    y, srcg, d0, d1 = _prep_call(x_pad, W1, src2d, dst2d)

    srcg_t = _to_slabs(srcg.reshape(-1), ZERO_ROW)
    dstl_t = jnp.stack([_to_slabs(d0.reshape(-1), DUMMY_DST),
                        _to_slabs(d1.reshape(-1), DUMMY_DST)])

    sc_step = _get_sc_step()
    h = sc_step(y, srcg_t, dstl_t)
    h = sc_step(h, srcg_t, dstl_t)
    h = _bn_call(h, b1.reshape(1, H), gamma.reshape(1, H), beta.reshape(1, H))
    h = sc_step(h, srcg_t, dstl_t)
    return _out_call(h, W2, b2.reshape(1, C))


# speedup vs baseline: 1.0105x; 1.0105x over previous
import functools

import jax
import jax.numpy as jnp
from jax import lax
from jax.experimental import pallas as pl
from jax.experimental.pallas import tpu as pltpu
from jax.experimental.pallas import tpu_sc as plsc

N = 10000
E = 320000
D = 128
H = 16
C = 64

NUM_CORES = 2
NUM_TILES = 16
NPAD = 10240
HALF = NPAD // NUM_CORES
ROWS_PER_TILE = HALF // NUM_TILES
ZERO_ROW = N
ACC_ROWS = HALF + 16
DUMMY_DST = HALF

EDGES_PER_TILE = E // NUM_TILES
CHUNK = 128
K_CHUNKS = 8
GROUP_E = K_CHUNKS * CHUNK
GROUPS = 20
NCHUNK = GROUPS * K_CHUNKS
EPT_PAD = NCHUNK * CHUNK


def _prep_body(x_ref, w1_ref, src_ref, dst_ref, y_ref, srcg_ref, d0_ref, d1_ref):
    x = x_ref[...]
    w1 = w1_ref[...]
    y_ref[...] = lax.dot_general(x, w1, (((1,), (1,)), ((), ())),
                                 preferred_element_type=jnp.float32,
                                 precision=lax.Precision.HIGHEST)
    src = src_ref[...]
    dst = dst_ref[...]
    srcg_ref[...] = jnp.where(src == dst, ZERO_ROW, src)
    d0_ref[...] = jnp.where(dst < HALF, dst, DUMMY_DST)
    d1_ref[...] = jnp.where(dst >= HALF, dst - HALF, DUMMY_DST)


_prep_call = pl.pallas_call(
    _prep_body,
    out_shape=(
        jax.ShapeDtypeStruct((NPAD, H), jnp.float32),
        jax.ShapeDtypeStruct((E // 128, 128), jnp.int32),
        jax.ShapeDtypeStruct((E // 128, 128), jnp.int32),
        jax.ShapeDtypeStruct((E // 128, 128), jnp.int32),
    ),
)


def _sc_step_body(h_hbm, srcg_hbm, dstl_hbm, out_hbm, acc, sidx, didx,
                  rows, gsem0, gsem1, ssem0, ssem1):
    gsem = (gsem0, gsem1)
    ssem = (ssem0, ssem1)
    c = lax.axis_index("c")
    s = lax.axis_index("s")
    base = c * HALF + s * ROWS_PER_TILE
    pltpu.sync_copy(h_hbm.at[pl.ds(base, ROWS_PER_TILE)],
                    acc.at[pl.ds(s * ROWS_PER_TILE, ROWS_PER_TILE)])
    pltpu.sync_copy(srcg_hbm.at[s], sidx)
    pltpu.sync_copy(dstl_hbm.at[c, s], didx)
    plsc.subcore_barrier()

    def issue_gather(g, p):
        pltpu.async_copy(h_hbm.at[sidx.at[pl.ds(g * GROUP_E, GROUP_E)]],
                         rows.at[p], gsem[p])

    def issue_scatter(g, p):
        pltpu.async_copy(rows.at[p], acc.at[didx.at[pl.ds(g * GROUP_E, GROUP_E)]],
                         ssem[p], add=True)

    def drain(sem, p):
        pltpu.make_async_copy(h_hbm.at[pl.ds(0, GROUP_E)], rows.at[p],
                              sem).wait()

    issue_gather(0, 0)

    def body(g2, carry):
        for p in (0, 1):
            g = 2 * g2 + p
            drain(gsem[p], p)
            issue_scatter(g, p)
            if p == 0:
                @pl.when(g2 > 0)
                def _():
                    drain(ssem[1], 1)
                issue_gather(2 * g2 + 1, 1)
            else:
                drain(ssem[0], 0)

                @pl.when(g2 < GROUPS // 2 - 1)
                def _():
                    issue_gather(2 * g2 + 2, 0)
        return carry

    lax.fori_loop(0, GROUPS // 2, body, 0)
    drain(ssem[1], 1)
    plsc.subcore_barrier()
    pltpu.sync_copy(acc.at[pl.ds(s * ROWS_PER_TILE, ROWS_PER_TILE)],
                    out_hbm.at[pl.ds(base, ROWS_PER_TILE)])


@functools.cache
def _get_sc_step():
    return pl.kernel(
        _sc_step_body,
        out_type=jax.ShapeDtypeStruct((NPAD, H), jnp.float32),
        mesh=plsc.VectorSubcoreMesh(core_axis_name="c", subcore_axis_name="s",
                                    num_cores=NUM_CORES, num_subcores=NUM_TILES),
        scratch_types=[
            pltpu.VMEM_SHARED((ACC_ROWS, H), jnp.float32),
            pltpu.VMEM((EPT_PAD,), jnp.int32),
            pltpu.VMEM((EPT_PAD,), jnp.int32),
            pltpu.VMEM((2, GROUP_E, H), jnp.float32),
            pltpu.SemaphoreType.DMA,
            pltpu.SemaphoreType.DMA,
            pltpu.SemaphoreType.DMA,
            pltpu.SemaphoreType.DMA,
        ],
        compiler_params=pltpu.CompilerParams(use_tc_tiling_on_sc=False),
    )


_SELU_SCALE = 1.0507009873554805
_SELU_ALPHA = 1.6732632423543772


def _bn_body(h_ref, b1_ref, g_ref, bt_ref, o_ref):
    h = h_ref[...]
    mask = (lax.broadcasted_iota(jnp.int32, (NPAD, 1), 0) < N).astype(jnp.float32)
    hb = (h + b1_ref[...]) * mask
    mean = jnp.sum(hb, axis=0, keepdims=True) / N
    ctr = (hb - mean) * mask
    var = jnp.sum(ctr * ctr, axis=0, keepdims=True) / N
    z = (hb - mean) * lax.rsqrt(var + 1e-5) * g_ref[...] + bt_ref[...]
    act = _SELU_SCALE * jnp.where(z > 0, z, _SELU_ALPHA * (jnp.exp(z) - 1.0))
    o_ref[...] = act * mask


_bn_call = pl.pallas_call(
    _bn_body,
    out_shape=jax.ShapeDtypeStruct((NPAD, H), jnp.float32),
)


def _out_body(h_ref, w2_ref, b2_ref, o_ref):
    h = h_ref[...]
    logits = lax.dot_general(h, w2_ref[...], (((1,), (1,)), ((), ())),
                             preferred_element_type=jnp.float32) + b2_ref[...]
    m = jnp.max(logits, axis=1, keepdims=True)
    e = jnp.exp(logits - m)
    p = e / jnp.sum(e, axis=1, keepdims=True)
    o_ref[...] = p[:N, :]


_out_call = pl.pallas_call(
    _out_body,
    out_shape=jax.ShapeDtypeStruct((N, C), jnp.float32),
)


def _to_slabs(a, fill):
    a = a.reshape(NUM_TILES, EDGES_PER_TILE)
    a = jnp.pad(a, ((0, 0), (0, EPT_PAD - EDGES_PER_TILE)), constant_values=fill)
    return a


def kernel(x, edge_index, W1, b1, gamma, beta, W2, b2):
    x_pad = jnp.pad(x, ((0, NPAD - N), (0, 0)))
    src2d = edge_index[0].reshape(E // 128, 128)
    dst2d = edge_index[1].reshape(E // 128, 128)
    y, srcg, d0, d1 = _prep_call(x_pad, W1, src2d, dst2d)

    srcg_t = _to_slabs(srcg.reshape(-1), ZERO_ROW)
    dstl_t = jnp.stack([_to_slabs(d0.reshape(-1), DUMMY_DST),
                        _to_slabs(d1.reshape(-1), DUMMY_DST)])

    sc_step = _get_sc_step()
    h = sc_step(y, srcg_t, dstl_t)
    h = sc_step(h, srcg_t, dstl_t)
    h = _bn_call(h, b1.reshape(1, H), gamma.reshape(1, H), beta.reshape(1, H))
    h = sc_step(h, srcg_t, dstl_t)
    return _out_call(h, W2, b2.reshape(1, C))


# speedup vs baseline: 1.8921x; 1.8724x over previous
import functools

import jax
import jax.numpy as jnp
from jax import lax
from jax.experimental import pallas as pl
from jax.experimental.pallas import tpu as pltpu
from jax.experimental.pallas import tpu_sc as plsc

N = 10000
E = 320000
D = 128
H = 16
C = 64

NUM_CORES = 2
NUM_TILES = 16
NUM_WORKERS = NUM_CORES * NUM_TILES
NPAD = 10240
ZERO_ROW = N
ACC_ROWS = NPAD + 16
DUMMY_DST = NPAD
INIT_ROWS = NPAD // NUM_TILES

EDGES_PER_WORKER = E // NUM_WORKERS
CHUNK = 128
NCHUNK = EDGES_PER_WORKER // CHUNK + 2
EPW_PAD = NCHUNK * CHUNK


def _prep_body(x_ref, w1_ref, src_ref, dst_ref, y_ref, srcg_ref):
    x = x_ref[...]
    w1 = w1_ref[...]
    y_ref[...] = lax.dot_general(x, w1, (((1,), (1,)), ((), ())),
                                 preferred_element_type=jnp.float32,
                                 precision=lax.Precision.HIGHEST)
    src = src_ref[...]
    dst = dst_ref[...]
    srcg_ref[...] = jnp.where(src == dst, ZERO_ROW, src)


_prep_call = pl.pallas_call(
    _prep_body,
    out_shape=(
        jax.ShapeDtypeStruct((NPAD, H), jnp.float32),
        jax.ShapeDtypeStruct((E // 128, 128), jnp.int32),
    ),
)


def _sc_step_body(h_hbm, z_hbm, srcg_hbm, dst_hbm, out_hbm, acc, sidx, didx,
                  rows, sem):
    c = lax.axis_index("c")
    s = lax.axis_index("s")
    w = c * NUM_TILES + s
    sl_init = pl.ds(s * INIT_ROWS, INIT_ROWS)

    @pl.when(c == 0)
    def _():
        pltpu.sync_copy(h_hbm.at[sl_init], acc.at[sl_init])

    @pl.when(c == 1)
    def _():
        pltpu.sync_copy(z_hbm.at[sl_init], acc.at[sl_init])

    pltpu.sync_copy(srcg_hbm.at[w], sidx)
    pltpu.sync_copy(dst_hbm.at[w], didx)
    plsc.subcore_barrier()

    def body(j, carry):
        sl = pl.ds(j * CHUNK, CHUNK)
        pltpu.async_copy(h_hbm.at[sidx.at[sl]], rows, sem).wait()
        pltpu.sync_copy(rows, acc.at[didx.at[sl]], add=True)
        return carry

    lax.fori_loop(0, NCHUNK, body, 0)
    plsc.subcore_barrier()
    pltpu.sync_copy(acc.at[sl_init], out_hbm.at[c, sl_init])


@functools.cache
def _get_sc_step():
    return pl.kernel(
        _sc_step_body,
        out_type=jax.ShapeDtypeStruct((NUM_CORES, NPAD, H), jnp.float32),
        mesh=plsc.VectorSubcoreMesh(core_axis_name="c", subcore_axis_name="s",
                                    num_cores=NUM_CORES, num_subcores=NUM_TILES),
        scratch_types=[
            pltpu.VMEM_SHARED((ACC_ROWS, H), jnp.float32),
            pltpu.VMEM((EPW_PAD,), jnp.int32),
            pltpu.VMEM((EPW_PAD,), jnp.int32),
            pltpu.VMEM((CHUNK, H), jnp.float32),
            pltpu.SemaphoreType.DMA,
        ],
        compiler_params=pltpu.CompilerParams(use_tc_tiling_on_sc=False),
    )


def _add_body(a_ref, o_ref):
    o_ref[...] = a_ref[0] + a_ref[1]


_add_call = pl.pallas_call(
    _add_body,
    out_shape=jax.ShapeDtypeStruct((NPAD, H), jnp.float32),
)


_SELU_SCALE = 1.0507009873554805
_SELU_ALPHA = 1.6732632423543772


def _bn_body(a_ref, b1_ref, g_ref, bt_ref, o_ref):
    h = a_ref[0] + a_ref[1]
    mask = (lax.broadcasted_iota(jnp.int32, (NPAD, 1), 0) < N).astype(jnp.float32)
    hb = (h + b1_ref[...]) * mask
    mean = jnp.sum(hb, axis=0, keepdims=True) / N
    ctr = (hb - mean) * mask
    var = jnp.sum(ctr * ctr, axis=0, keepdims=True) / N
    z = (hb - mean) * lax.rsqrt(var + 1e-5) * g_ref[...] + bt_ref[...]
    act = _SELU_SCALE * jnp.where(z > 0, z, _SELU_ALPHA * (jnp.exp(z) - 1.0))
    o_ref[...] = act * mask


_bn_call = pl.pallas_call(
    _bn_body,
    out_shape=jax.ShapeDtypeStruct((NPAD, H), jnp.float32),
)


def _out_body(a_ref, w2_ref, b2_ref, o_ref):
    h = a_ref[0] + a_ref[1]
    logits = lax.dot_general(h, w2_ref[...], (((1,), (1,)), ((), ())),
                             preferred_element_type=jnp.float32) + b2_ref[...]
    m = jnp.max(logits, axis=1, keepdims=True)
    e = jnp.exp(logits - m)
    p = e / jnp.sum(e, axis=1, keepdims=True)
    o_ref[...] = p[:N, :]


_out_call = pl.pallas_call(
    _out_body,
    out_shape=jax.ShapeDtypeStruct((N, C), jnp.float32),
)


def _to_slabs(a, fill):
    a = a.reshape(NUM_WORKERS, EDGES_PER_WORKER)
    return jnp.pad(a, ((0, 0), (0, EPW_PAD - EDGES_PER_WORKER)),
                   constant_values=fill)


def kernel(x, edge_index, W1, b1, gamma, beta, W2, b2):
    x_pad = jnp.pad(x, ((0, NPAD - N), (0, 0)))
    src2d = edge_index[0].reshape(E // 128, 128)
    dst2d = edge_index[1].reshape(E // 128, 128)
    y, srcg = _prep_call(x_pad, W1, src2d, dst2d)

    srcg_t = _to_slabs(srcg.reshape(-1), ZERO_ROW)
    dst_t = _to_slabs(edge_index[1], DUMMY_DST)
    z = jnp.zeros((NPAD, H), jnp.float32)

    sc_step = _get_sc_step()
    a = sc_step(y, z, srcg_t, dst_t)
    h = _add_call(a)
    a = sc_step(h, z, srcg_t, dst_t)
    h = _bn_call(a, b1.reshape(1, H), gamma.reshape(1, H), beta.reshape(1, H))
    a = sc_step(h, z, srcg_t, dst_t)
    return _out_call(a, W2, b2.reshape(1, C))
